# Initial kernel scaffold; baseline (speedup 1.0000x reference)
import jax
import jax.numpy as jnp
from jax.experimental import pallas as pl


def kernel(observations, lambda_r, Phi_f, Phi_h, mu, sigma2, chol_Qh):
    raise NotImplementedError("write your pallas kernel here")


# speedup vs baseline: 5.6125x; 5.6125x over previous
import jax
import jax.numpy as jnp
from jax.experimental import pallas as pl
from jax.experimental.pallas import tpu as pltpu

_P = 8192
_T = 256
_K = 5
_NOBS = 100
_SEED = 42
_THRESH = 0.5 * _P
_PB = 512


def _gen_noise(mu, chol_Qh):
    key = jax.random.PRNGKey(_SEED)
    key, k1, k2 = jax.random.split(key, 3)
    f0 = jax.random.normal(k1, (_K, _P), dtype=jnp.float32)
    h0 = mu[:, None] + chol_Qh @ jax.random.normal(k2, (_K, _P), dtype=jnp.float32)

    def step(key_curr, _):
        key_next, kh, kf = jax.random.split(key_curr, 3)
        eps_h = jax.random.normal(kh, (_K, _P), dtype=jnp.float32)
        eps_f = jax.random.normal(kf, (_K, _P), dtype=jnp.float32)
        ku = jax.random.split(key_next)[1]
        u = jax.random.uniform(ku)
        return key_next, (eps_h, eps_f, u)

    _, (eps_h_all, eps_f_all, u_all) = jax.lax.scan(step, key, None, length=_T)
    return f0, h0, eps_h_all, eps_f_all, u_all


def _body(obs_ref, lam_ref, phif_ref, phih_ref, mu_ref, s2_ref, lt_ref,
          chol_ref, f0_ref, h0_ref, epsh_ref, epsf_ref, u_ref,
          mean_ref, cov_ref, ll_ref,
          f_s, h_s, lw_s, acc_s):
    t = pl.program_id(0)

    @pl.when(t == 0)
    def _init():
        f_s[...] = f0_ref[...]
        h_s[...] = h0_ref[...]
        lw_s[...] = jnp.full((1, _P), -jnp.log(float(_P)), jnp.float32)
        acc_s[...] = jnp.zeros((1, 1), jnp.float32)

    mu = mu_ref[...]
    obs = obs_ref[0]
    eps_h = epsh_ref[0]
    eps_f = epsf_ref[0]

    h = h_s[...]
    f = f_s[...]
    h_next = (mu + jnp.dot(phih_ref[...], h - mu, preferred_element_type=jnp.float32)
              + jnp.dot(chol_ref[...], eps_h, preferred_element_type=jnp.float32))
    f_next = (jnp.dot(phif_ref[...], f, preferred_element_type=jnp.float32)
              + jnp.exp(0.5 * h_next) * eps_f)

    pred = jnp.dot(lam_ref[...], f_next, preferred_element_type=jnp.float32)
    resid = obs - pred
    ll_row = -0.5 * jnp.sum(resid * resid / s2_ref[...] + lt_ref[...],
                            axis=0, keepdims=True)

    un = lw_s[...] + ll_row

    m = jnp.max(un)
    s1 = jnp.sum(jnp.exp(un - m))
    lse = jnp.log(s1) + m
    acc_s[...] = acc_s[...] + lse
    ll_ref[...] = acc_s[...]

    a2 = 2.0 * (un - lse)
    m2 = jnp.max(a2)
    ess = jnp.exp(-(jnp.log(jnp.sum(jnp.exp(a2 - m2))) + m2))

    w_row = jnp.exp(un - lse)

    p10 = jnp.concatenate([f_next, h_next], axis=0)
    mean_row = jax.lax.dot_general(w_row, p10, (((1,), (1,)), ((), ())),
                                   preferred_element_type=jnp.float32)
    diff = p10 - jax.lax.dot_general(p10, w_row, (((1,), (1,)), ((), ())),
                                     preferred_element_type=jnp.float32)
    dw = diff * w_row
    cov = jax.lax.dot_general(dw, diff, (((1,), (1,)), ((), ())),
                              preferred_element_type=jnp.float32)
    mean_ref[...] = mean_row[None]
    cov_ref[...] = cov[None]

    do_rs = ess < _THRESH

    @pl.when(do_rs)
    def _resample():
        jiota = jax.lax.broadcasted_iota(jnp.int32, (_P, 1), 0)
        cum = jnp.zeros((_P, 1), jnp.float32)
        for b in range(_P // _PB):
            kk = jax.lax.broadcasted_iota(jnp.int32, (1, _PB), 1) + b * _PB
            tri = (jiota >= kk).astype(jnp.float32)
            wb = w_row[:, b * _PB:(b + 1) * _PB]
            cum = cum + jax.lax.dot_general(
                tri, wb, (((1,), (1,)), ((), ())),
                preferred_element_type=jnp.float32)
        cum = jnp.where(jiota == _P - 1, jnp.float32(3.0e38), cum)

        u = u_ref[0, 0, 0]
        d10 = p10 - jnp.concatenate(
            [p10[:, 1:], jnp.zeros((10, 1), jnp.float32)], axis=1)
        blocks = []
        for b in range(_P // _PB):
            lane_i = jax.lax.broadcasted_iota(jnp.int32, (1, _PB), 1).astype(jnp.float32)
            pos = (lane_i + (jnp.float32(b * _PB) + u)) * jnp.float32(1.0 / _P)
            z = (pos <= cum).astype(jnp.float32)
            blocks.append(jnp.dot(d10, z, preferred_element_type=jnp.float32))
        out10 = jnp.concatenate(blocks, axis=1)
        f_s[...] = out10[0:_K]
        h_s[...] = out10[_K:2 * _K]
        lw_s[...] = jnp.full((1, _P), -jnp.log(float(_P)), jnp.float32)

    @pl.when(jnp.logical_not(do_rs))
    def _keep():
        f_s[...] = f_next
        h_s[...] = h_next
        lw_s[...] = un


def kernel(observations, lambda_r, Phi_f, Phi_h, mu, sigma2, chol_Qh):
    f0, h0, eps_h_all, eps_f_all, u_all = _gen_noise(mu, chol_Qh)
    obs3 = observations[:, :, None]
    s2c = sigma2[:, None]
    ltc = jnp.log(2.0 * jnp.pi * sigma2)[:, None]
    mu_c = mu[:, None]
    u2 = u_all[:, None, None]

    grid = (_T,)
    means, covs, ll = pl.pallas_call(
        _body,
        grid=grid,
        in_specs=[
            pl.BlockSpec((1, _NOBS, 1), lambda t: (t, 0, 0)),
            pl.BlockSpec((_NOBS, _K), lambda t: (0, 0)),
            pl.BlockSpec((_K, _K), lambda t: (0, 0)),
            pl.BlockSpec((_K, _K), lambda t: (0, 0)),
            pl.BlockSpec((_K, 1), lambda t: (0, 0)),
            pl.BlockSpec((_NOBS, 1), lambda t: (0, 0)),
            pl.BlockSpec((_NOBS, 1), lambda t: (0, 0)),
            pl.BlockSpec((_K, _K), lambda t: (0, 0)),
            pl.BlockSpec((_K, _P), lambda t: (0, 0)),
            pl.BlockSpec((_K, _P), lambda t: (0, 0)),
            pl.BlockSpec((1, _K, _P), lambda t: (t, 0, 0)),
            pl.BlockSpec((1, _K, _P), lambda t: (t, 0, 0)),
            pl.BlockSpec((1, 1, 1), lambda t: (t, 0, 0)),
        ],
        out_specs=[
            pl.BlockSpec((1, 1, 2 * _K), lambda t: (t, 0, 0)),
            pl.BlockSpec((1, 2 * _K, 2 * _K), lambda t: (t, 0, 0)),
            pl.BlockSpec((1, 1), lambda t: (0, 0)),
        ],
        out_shape=[
            jax.ShapeDtypeStruct((_T, 1, 2 * _K), jnp.float32),
            jax.ShapeDtypeStruct((_T, 2 * _K, 2 * _K), jnp.float32),
            jax.ShapeDtypeStruct((1, 1), jnp.float32),
        ],
        scratch_shapes=[
            pltpu.VMEM((_K, _P), jnp.float32),
            pltpu.VMEM((_K, _P), jnp.float32),
            pltpu.VMEM((1, _P), jnp.float32),
            pltpu.VMEM((1, 1), jnp.float32),
        ],
    )(obs3, lambda_r, Phi_f, Phi_h, mu_c, s2c, ltc, chol_Qh, f0, h0,
      eps_h_all, eps_f_all, u2)
    return means[:, 0, :], covs, ll[0, 0]
